# baseline (speedup 1.0000x reference)
import functools

import jax
import jax.numpy as jnp
from jax import lax
from jax.experimental import pallas as pl
from jax.experimental.pallas import tpu as pltpu
from jax.experimental.pallas import tpu_sc as plsc

BATCH = 4
SEQ = 2048
EMBED_DIM = 768
NUM_CORES = 2
NUM_SUBCORES = 16
NUM_WORKERS = NUM_CORES * NUM_SUBCORES
S_PER_W = SEQ // NUM_WORKERS
SCH = 16
UNITS = S_PER_W // SCH
ROWS = SCH * BATCH
NSET = 2
LANES = 16
VECS_PER_ROW = EMBED_DIM // LANES
NUM_UNITS_G = SEQ // SCH


def _embed_kernel(xg_hbm, oidx_hbm, tok_hbm, pos_hbm, out_hbm,
                  idx_v, oidx_v, tok0, tok1, pos0, pos1,
                  g0, g1, p0, p1, s0sem, s1sem):
    wid = lax.axis_index("s") * NUM_CORES + lax.axis_index("c")
    g_base = wid * UNITS

    tok_bufs = (tok0, tok1)
    pos_bufs = (pos0, pos1)
    gsems = (g0, g1)
    psems = (p0, p1)
    ssems = (s0sem, s1sem)

    pltpu.sync_copy(xg_hbm.at[pl.ds(g_base, UNITS)], idx_v)
    pltpu.sync_copy(oidx_hbm.at[pl.ds(g_base, UNITS)], oidx_v)

    def fetch(u):
        st = u % NSET
        g = pltpu.async_copy(tok_hbm.at[idx_v.at[u]], tok_bufs[st], gsems[st])
        p = pltpu.async_copy(pos_hbm.at[pl.ds((g_base + u) * SCH, SCH)],
                             pos_bufs[st], psems[st])
        return g, p

    fetches = [None] * UNITS
    stores = [None] * UNITS
    fetches[0] = fetch(0)
    fetches[1] = fetch(1)

    for u in range(UNITS):
        st = u % NSET
        for cp in fetches[u]:
            cp.wait()
        buf = tok_bufs[st]
        posb = pos_bufs[st]

        def add_row(i, _):
            for j in range(VECS_PER_ROW):
                sl = pl.ds(j * LANES, LANES)
                pv = posb[i, sl]
                for b in range(BATCH):
                    buf[b * SCH + i, sl] = buf[b * SCH + i, sl] + pv
            return _

        lax.fori_loop(0, SCH, add_row, None)

        stores[u] = pltpu.async_copy(buf, out_hbm.at[oidx_v.at[u]], ssems[st])
        if u + 2 < UNITS:
            stores[u].wait()
            fetches[u + 2] = fetch(u + 2)

    for u in range(UNITS - 2, UNITS):
        stores[u].wait()


@jax.jit
def _embed(x, tok_table, pos_table):
    x_g = jnp.transpose(x.reshape(BATCH, NUM_UNITS_G, SCH), (1, 0, 2))
    x_g = x_g.reshape(NUM_UNITS_G, ROWS)
    oidx = (jnp.arange(BATCH, dtype=jnp.int32)[None, :, None] * SEQ
            + (jnp.arange(NUM_UNITS_G, dtype=jnp.int32) * SCH)[:, None, None]
            + jnp.arange(SCH, dtype=jnp.int32)[None, None, :])
    oidx = oidx.reshape(NUM_UNITS_G, ROWS)

    mesh = plsc.VectorSubcoreMesh(core_axis_name="c", subcore_axis_name="s")
    kfn = functools.partial(
        pl.kernel,
        mesh=mesh,
        out_type=jax.ShapeDtypeStruct((BATCH * SEQ, EMBED_DIM), jnp.float32),
        scratch_types=[
            pltpu.VMEM((UNITS, ROWS), jnp.int32),
            pltpu.VMEM((UNITS, ROWS), jnp.int32),
            pltpu.VMEM((ROWS, EMBED_DIM), jnp.float32),
            pltpu.VMEM((ROWS, EMBED_DIM), jnp.float32),
            pltpu.VMEM((SCH, EMBED_DIM), jnp.float32),
            pltpu.VMEM((SCH, EMBED_DIM), jnp.float32),
            pltpu.SemaphoreType.DMA,
            pltpu.SemaphoreType.DMA,
            pltpu.SemaphoreType.DMA,
            pltpu.SemaphoreType.DMA,
            pltpu.SemaphoreType.DMA,
            pltpu.SemaphoreType.DMA,
        ],
    )(_embed_kernel)
    out = kfn(x_g, oidx, tok_table, pos_table)
    return out.reshape(BATCH, SEQ, EMBED_DIM)


def kernel(x, tok_table, pos_table):
    return _embed(x, tok_table, pos_table)


# speedup vs baseline: 1.1749x; 1.1749x over previous
import functools

import jax
import jax.numpy as jnp
from jax import lax
from jax.experimental import pallas as pl
from jax.experimental.pallas import tpu as pltpu
from jax.experimental.pallas import tpu_sc as plsc

BATCH = 4
SEQ = 2048
EMBED_DIM = 768
NUM_CORES = 2
NUM_SUBCORES = 16
NUM_WORKERS = NUM_CORES * NUM_SUBCORES
S_PER_W = SEQ // NUM_WORKERS
ROWS = 32
UNITS_PER_B = S_PER_W // ROWS
UNITS = BATCH * UNITS_PER_B
LANES = 16
VECS_PER_ROW = EMBED_DIM // LANES
NBUF = 3


def _embed_kernel(x_hbm, tok_hbm, pos_hbm, out_hbm,
                  idx_v, pos_v, tok0, tok1, tok2,
                  psem, g0, g1, g2, s0sem, s1sem, s2sem):
    wid = lax.axis_index("s") * NUM_CORES + lax.axis_index("c")
    s0 = wid * S_PER_W

    tok_bufs = (tok0, tok1, tok2)
    gsems = (g0, g1, g2)
    ssems = (s0sem, s1sem, s2sem)

    for b in range(BATCH):
        pltpu.sync_copy(x_hbm.at[b, pl.ds(s0, S_PER_W)], idx_v.at[b])

    pos_cp = pltpu.async_copy(pos_hbm.at[pl.ds(s0, S_PER_W)], pos_v, psem)

    def gather(u):
        b, half = divmod(u, UNITS_PER_B)
        return pltpu.async_copy(
            tok_hbm.at[idx_v.at[b, pl.ds(half * ROWS, ROWS)]],
            tok_bufs[u % NBUF], gsems[u % NBUF])

    gathers = [None] * UNITS
    stores = [None] * UNITS
    gathers[0] = gather(0)
    pos_cp.wait()

    for u in range(UNITS):
        if u + 1 < UNITS:
            if u >= 2:
                stores[u - 2].wait()
            gathers[u + 1] = gather(u + 1)
        gathers[u].wait()

        buf = tok_bufs[u % NBUF]
        b, half = divmod(u, UNITS_PER_B)
        off = half * ROWS

        def add_row(r, _):
            for j in range(VECS_PER_ROW):
                sl = pl.ds(j * LANES, LANES)
                plsc.addupdate(buf.at[r, sl], pos_v[off + r, sl])
            return _

        lax.fori_loop(0, ROWS, add_row, None)

        stores[u] = pltpu.async_copy(
            buf, out_hbm.at[b, pl.ds(s0 + off, ROWS)], ssems[u % NBUF])

    for u in range(UNITS - 3, UNITS):
        stores[u].wait()


@jax.jit
def _embed(x, tok_table, pos_table):
    mesh = plsc.VectorSubcoreMesh(core_axis_name="c", subcore_axis_name="s")
    kfn = functools.partial(
        pl.kernel,
        mesh=mesh,
        out_type=jax.ShapeDtypeStruct((BATCH, SEQ, EMBED_DIM), jnp.float32),
        scratch_types=[
            pltpu.VMEM((BATCH, S_PER_W), jnp.int32),
            pltpu.VMEM((S_PER_W, EMBED_DIM), jnp.float32),
            pltpu.VMEM((ROWS, EMBED_DIM), jnp.float32),
            pltpu.VMEM((ROWS, EMBED_DIM), jnp.float32),
            pltpu.VMEM((ROWS, EMBED_DIM), jnp.float32),
            pltpu.SemaphoreType.DMA,
            pltpu.SemaphoreType.DMA,
            pltpu.SemaphoreType.DMA,
            pltpu.SemaphoreType.DMA,
            pltpu.SemaphoreType.DMA,
            pltpu.SemaphoreType.DMA,
            pltpu.SemaphoreType.DMA,
        ],
    )(_embed_kernel)
    return kfn(x, tok_table, pos_table)


def kernel(x, tok_table, pos_table):
    return _embed(x, tok_table, pos_table)


# speedup vs baseline: 1.3443x; 1.1442x over previous
import functools

import jax
import jax.numpy as jnp
from jax import lax
from jax.experimental import pallas as pl
from jax.experimental.pallas import tpu as pltpu
from jax.experimental.pallas import tpu_sc as plsc

BATCH = 4
SEQ = 2048
EMBED_DIM = 768
NUM_CORES = 2
NUM_SUBCORES = 16
NUM_WORKERS = NUM_CORES * NUM_SUBCORES
S_PER_W = SEQ // NUM_WORKERS
ROWS = 32
UNITS_PER_B = S_PER_W // ROWS
UNITS = BATCH * UNITS_PER_B
LANES = 16
VECS_PER_ROW = EMBED_DIM // LANES
NBUF = 3


def _embed_kernel(x_hbm, tok_hbm, pos_hbm, out_hbm,
                  idx_v, pos_v, tok0, tok1, tok2,
                  psem, g0, g1, g2, s0sem, s1sem, s2sem):
    wid = lax.axis_index("s") * NUM_CORES + lax.axis_index("c")
    s0 = wid * S_PER_W

    tok_bufs = (tok0, tok1, tok2)
    gsems = (g0, g1, g2)
    ssems = (s0sem, s1sem, s2sem)

    pos_cp = pltpu.async_copy(pos_hbm.at[pl.ds(s0, S_PER_W)], pos_v, psem)

    def gather(u):
        b, half = divmod(u, UNITS_PER_B)
        return pltpu.async_copy(
            tok_hbm.at[idx_v.at[b, pl.ds(half * ROWS, ROWS)]],
            tok_bufs[u % NBUF], gsems[u % NBUF])

    pltpu.sync_copy(x_hbm.at[0, pl.ds(s0, S_PER_W)], idx_v.at[0])
    gathers = [None] * UNITS
    stores = [None] * UNITS
    gathers[0] = gather(0)
    for b in range(1, BATCH):
        pltpu.sync_copy(x_hbm.at[b, pl.ds(s0, S_PER_W)], idx_v.at[b])
    pos_cp.wait()

    for u in range(UNITS):
        if u + 1 < UNITS:
            if u >= 2:
                stores[u - 2].wait()
            gathers[u + 1] = gather(u + 1)
        gathers[u].wait()

        buf = tok_bufs[u % NBUF]
        b, half = divmod(u, UNITS_PER_B)
        off = half * ROWS

        @plsc.parallel_loop(0, ROWS, 1, unroll=2)
        def add_row(r):
            for j in range(VECS_PER_ROW):
                sl = pl.ds(j * LANES, LANES)
                plsc.addupdate(buf.at[r, sl], pos_v[off + r, sl])

        stores[u] = pltpu.async_copy(
            buf, out_hbm.at[b, pl.ds(s0 + off, ROWS)], ssems[u % NBUF])

    for u in range(UNITS - 3, UNITS):
        stores[u].wait()


@jax.jit
def _embed(x, tok_table, pos_table):
    mesh = plsc.VectorSubcoreMesh(core_axis_name="c", subcore_axis_name="s")
    kfn = functools.partial(
        pl.kernel,
        mesh=mesh,
        out_type=jax.ShapeDtypeStruct((BATCH, SEQ, EMBED_DIM), jnp.float32),
        scratch_types=[
            pltpu.VMEM((BATCH, S_PER_W), jnp.int32),
            pltpu.VMEM((S_PER_W, EMBED_DIM), jnp.float32),
            pltpu.VMEM((ROWS, EMBED_DIM), jnp.float32),
            pltpu.VMEM((ROWS, EMBED_DIM), jnp.float32),
            pltpu.VMEM((ROWS, EMBED_DIM), jnp.float32),
            pltpu.SemaphoreType.DMA,
            pltpu.SemaphoreType.DMA,
            pltpu.SemaphoreType.DMA,
            pltpu.SemaphoreType.DMA,
            pltpu.SemaphoreType.DMA,
            pltpu.SemaphoreType.DMA,
            pltpu.SemaphoreType.DMA,
        ],
    )(_embed_kernel)
    return kfn(x, tok_table, pos_table)


def kernel(x, tok_table, pos_table):
    return _embed(x, tok_table, pos_table)
